# baseline (speedup 1.0000x reference)
import functools

import jax
import jax.numpy as jnp
from jax import lax
from jax.experimental import pallas as pl
from jax.experimental.pallas import tpu as pltpu
from jax.experimental.pallas import tpu_sc as plsc

N_CORES = 2
N_SUBCORES = 16
NW = N_CORES * N_SUBCORES
CHUNK = 128


def _sc_gather(table, idx2d, total_rows):
    n, c = table.shape
    _, cnt, _ = idx2d.shape
    outer = (cnt + 3) // 3
    mesh = plsc.VectorSubcoreMesh(core_axis_name="c", subcore_axis_name="s")

    @functools.partial(
        pl.kernel,
        mesh=mesh,
        out_type=jax.ShapeDtypeStruct((total_rows, c), jnp.float32),
        scratch_types=[
            pltpu.VMEM((cnt, CHUNK), jnp.int32),
            pltpu.VMEM((3, CHUNK, c), jnp.float32),
            pltpu.SemaphoreType.DMA,
            pltpu.SemaphoreType.DMA,
            pltpu.SemaphoreType.DMA,
            pltpu.SemaphoreType.DMA,
            pltpu.SemaphoreType.DMA,
            pltpu.SemaphoreType.DMA,
        ],
    )
    def k(table_hbm, idx_hbm, out_hbm, idx_all, rows_v, sg0, sg1, sg2, sw0, sw1, sw2):
        sg = [sg0, sg1, sg2]
        sw = [sw0, sw1, sw2]
        wid = lax.axis_index("s") * N_CORES + lax.axis_index("c")
        start = wid * cnt
        pltpu.sync_copy(idx_hbm.at[wid], idx_all)
        pltpu.async_copy(table_hbm.at[idx_all.at[0]], rows_v.at[0], sg[0])
        pltpu.async_copy(table_hbm.at[idx_all.at[1]], rows_v.at[1], sg[1])

        def outer_step(jj, carry):
            for b in range(3):
                j = jj * 3 + b
                bn = (b + 2) % 3

                @pl.when((j >= 1) & (j <= cnt))
                def _():
                    pltpu.make_async_copy(
                        rows_v.at[bn], out_hbm.at[pl.ds(0, CHUNK)], sw[bn]
                    ).wait()

                @pl.when(j + 2 < cnt)
                def _():
                    pltpu.async_copy(
                        table_hbm.at[idx_all.at[j + 2]], rows_v.at[bn], sg[bn])

                @pl.when(j < cnt)
                def _():
                    pltpu.make_async_copy(
                        table_hbm.at[idx_all.at[j]], rows_v.at[b], sg[b]
                    ).wait()
                    pltpu.async_copy(
                        rows_v.at[b],
                        out_hbm.at[pl.ds((start + j) * CHUNK, CHUNK)], sw[b])

            return carry

        lax.fori_loop(0, outer, outer_step, 0)

    return k(table, idx2d)


def _tc_combine(gathered, guid2, w2d, m, kk, c, cmid, bm, grp):
    ng = bm // grp
    rg = grp * kk
    dg = grp * cmid

    def body(g_ref, guid_ref, w_ref, out_ref):
        row8 = lax.broadcasted_iota(jnp.int32, (8, c), 0)
        col8 = lax.broadcasted_iota(jnp.int32, (8, c), 1)
        exp = (col8 // (c // 8) == row8).astype(jnp.float32)
        guid2 = guid_ref[...].reshape(bm * kk, 8)
        gexp = jnp.dot(guid2, exp, preferred_element_type=jnp.float32)
        g3 = g_ref[...] * gexp
        t2r = lax.broadcasted_iota(jnp.int32, (cmid, dg), 0)
        t2c = lax.broadcasted_iota(jnp.int32, (cmid, dg), 1)
        t2 = (t2r == t2c % cmid).astype(jnp.float32)
        mkr = lax.broadcasted_iota(jnp.int32, (rg, dg), 0)
        mkc = lax.broadcasted_iota(jnp.int32, (rg, dg), 1)
        mask_t = (mkr // kk == mkc // cmid).astype(jnp.float32)
        w2 = w_ref[...].reshape(bm * kk, cmid)
        for gg in range(ng):
            g3g = g3[gg * rg : (gg + 1) * rg, :]
            w3g = w2[gg * rg : (gg + 1) * rg, :]
            w2t = jnp.dot(w3g, t2, preferred_element_type=jnp.float32) * mask_t
            out_g = lax.dot_general(
                w2t, g3g, (((0,), (0,)), ((), ())),
                preferred_element_type=jnp.float32)
            out_ref[gg * grp : (gg + 1) * grp] = out_g.reshape(grp, cmid, c)

    grid = (m // bm,)
    return pl.pallas_call(
        body,
        grid=grid,
        in_specs=[
            pl.BlockSpec((bm * kk, c), lambda i: (i, 0)),
            pl.BlockSpec((1, bm, kk, 8), lambda i: (0, i, 0, 0)),
            pl.BlockSpec((1, bm, kk, cmid), lambda i: (0, i, 0, 0)),
        ],
        out_specs=pl.BlockSpec((bm, cmid, c), lambda i: (i, 0, 0)),
        out_shape=jax.ShapeDtypeStruct((m, cmid, c), jnp.float32),
    )(gathered, guid2, w2d)


def kernel(input_features, neighbor_inds, guidance, weightnet):
    b, n, c = input_features.shape
    _, m, kk = neighbor_inds.shape
    h = guidance.shape[-1]
    cmid = weightnet.shape[-1]
    assert b == 1

    table = input_features[0]
    nch = -(-(m * kk) // CHUNK)
    per_w = -(-nch // NW)
    per_w += (-per_w) % 8
    idx_flat = neighbor_inds.reshape(-1)
    idx3 = jnp.pad(idx_flat, (0, NW * per_w * CHUNK - m * kk)).reshape(
        NW, per_w, CHUNK)
    gathered = _sc_gather(table, idx3, NW * per_w * CHUNK)

    guid3 = guidance
    w3d = weightnet

    out_t3 = _tc_combine(gathered, guid3, w3d, m, kk, c, cmid, bm=80, grp=8)
    return jnp.swapaxes(out_t3, 1, 2).reshape(b, m, c * cmid)
